# baseline (speedup 1.0000x reference)
import functools

import jax
import jax.numpy as jnp
from jax import lax
from jax.experimental import pallas as pl
from jax.experimental.pallas import tpu as pltpu
from jax.experimental.pallas import tpu_sc as plsc

VOCAB = 100000
DIM = 128
BATCH = 16384

_info = plsc.get_sparse_core_info()
_NC, _NS, _L = _info.num_cores, _info.num_subcores, _info.num_lanes
_NW = _NC * _NS
_CHUNK = BATCH // _NW
_NV = _CHUNK // _L
_Q = 32
_QSH = _Q.bit_length() - 1
_PAD = _CHUNK + 3 * _Q
_NQ = _PAD // _Q


def _body(t0, t1, t2, ids_hbm, split_hbm, out_hbm,
          idx_v, spl_v, sidx_v, spos_v, rows_v, gsem, ssem):
    wid = lax.axis_index("s") * _NC + lax.axis_index("c")
    base = wid * _CHUNK
    pltpu.sync_copy(ids_hbm.at[pl.ds(base, _CHUNK)], idx_v)
    pltpu.sync_copy(split_hbm.at[pl.ds(base, _CHUNK)], spl_v)

    iota = lax.iota(jnp.int32, _L)

    def count(i, c):
        s = spl_v[pl.ds(i * _L, _L)]
        a0, a1 = c
        return (a0 + (s == 0).astype(jnp.int32), a1 + (s == 1).astype(jnp.int32))

    zz = jnp.zeros((_L,), jnp.int32)
    a0, a1 = lax.fori_loop(0, _NV, count, (zz, zz))
    c0 = jnp.sum(a0)
    c1 = jnp.sum(a1)
    c2 = _CHUNK - c0 - c1
    b1 = (c0 + _Q - 1) & ~(_Q - 1)
    b2 = b1 + ((c1 + _Q - 1) & ~(_Q - 1))
    b3 = b2 + ((c2 + _Q - 1) & ~(_Q - 1))

    def place(i, o):
        o0, o1, o2 = o
        s = spl_v[pl.ds(i * _L, _L)]
        ids = idx_v[pl.ds(i * _L, _L)]
        pos = base + i * _L + iota
        m0 = (s == 0).astype(jnp.int32)
        m1 = (s == 1).astype(jnp.int32)
        m2 = (s == 2).astype(jnp.int32)
        r0 = plsc.cumsum(m0)
        r1 = plsc.cumsum(m1)
        r2 = plsc.cumsum(m2)
        slot = (m0 * (o0 + r0 - 1) + m1 * (o1 + r1 - 1) + m2 * (o2 + r2 - 1))
        plsc.store_scatter(sidx_v, [slot], ids)
        plsc.store_scatter(spos_v, [slot >> _QSH, slot & (_Q - 1)], pos)
        return (o0 + r0[_L - 1], o1 + r1[_L - 1], o2 + r2[_L - 1])

    o0, o1, o2 = lax.fori_loop(0, _NV, place, (jnp.int32(0), b1, b2))

    for ot, bt, bn in ((o0, jnp.int32(0), b1), (o1, b1, b2), (o2, b2, b3)):
        first_idx = sidx_v[pl.ds(bt, _L)][0]
        first_pos = spos_v[bt >> _QSH, pl.ds(0, _L)][0]
        fi = jnp.full((_L,), first_idx, jnp.int32)
        fp = jnp.full((_L,), first_pos, jnp.int32)
        for j in range(_Q // _L):
            slots = ot + j * _L + iota
            mask = slots < bn
            plsc.store_scatter(sidx_v, [slots], fi, mask=mask)
            plsc.store_scatter(spos_v, [slots >> _QSH, slots & (_Q - 1)],
                               fp, mask=mask)

    for tab, qs, qe in ((t0, jnp.int32(0), b1 // _Q), (t1, b1 // _Q, b2 // _Q),
                        (t2, b2 // _Q, b3 // _Q)):
        def fire(q, _, tab=tab):
            pltpu.async_copy(tab.at[sidx_v.at[pl.ds(q * _Q, _Q)]],
                             rows_v.at[pl.ds(q * _Q, _Q)], gsem)
            return _
        lax.fori_loop(qs, qe, fire, 0)

    def drain_g(q, _):
        pltpu.make_async_copy(t0.at[sidx_v.at[pl.ds(0, _Q)]],
                              rows_v.at[pl.ds(0, _Q)], gsem).wait()
        return _
    lax.fori_loop(0, b3 // _Q, drain_g, 0)

    def fire_s(q, _):
        pltpu.async_copy(rows_v.at[pl.ds(q * _Q, _Q)],
                         out_hbm.at[spos_v.at[q]], ssem)
        return _
    lax.fori_loop(0, b3 // _Q, fire_s, 0)

    def drain_s(q, _):
        pltpu.make_async_copy(rows_v.at[pl.ds(0, _Q)],
                              out_hbm.at[spos_v.at[0]], ssem).wait()
        return _
    lax.fori_loop(0, b3 // _Q, drain_s, 0)


@jax.jit
def _run(t0, t1, t2, ids, split):
    mesh = plsc.VectorSubcoreMesh(core_axis_name="c", subcore_axis_name="s")
    return pl.kernel(
        _body,
        mesh=mesh,
        compiler_params=pltpu.CompilerParams(
            needs_layout_passes=False,
            disable_bounds_checks=True,
            disable_semaphore_checks=True,
        ),
        out_type=jax.ShapeDtypeStruct((BATCH, DIM), jnp.float32),
        scratch_types=[
            pltpu.VMEM((_CHUNK,), jnp.int32),
            pltpu.VMEM((_CHUNK,), jnp.int32),
            pltpu.VMEM((_PAD,), jnp.int32),
            pltpu.VMEM((_NQ, _Q), jnp.int32),
            pltpu.VMEM((_PAD, DIM), jnp.float32),
            pltpu.SemaphoreType.DMA,
            pltpu.SemaphoreType.DMA,
        ],
    )(t0, t1, t2, ids, split)


def kernel(query_enc_train, query_enc_dev, query_enc_test, ex_ids, split):
    return _run(query_enc_train, query_enc_dev, query_enc_test,
                ex_ids.astype(jnp.int32), split.astype(jnp.int32))


# speedup vs baseline: 1.1317x; 1.1317x over previous
import functools

import jax
import jax.numpy as jnp
from jax import lax
from jax.experimental import pallas as pl
from jax.experimental.pallas import tpu as pltpu
from jax.experimental.pallas import tpu_sc as plsc

VOCAB = 100000
DIM = 128
BATCH = 16384

_info = plsc.get_sparse_core_info()
_NC, _NS, _L = _info.num_cores, _info.num_subcores, _info.num_lanes
_NW = _NC * _NS
_CHUNK = BATCH // _NW
_NV = _CHUNK // _L
_Q = 16
_QSH = _Q.bit_length() - 1
_PAD = _CHUNK + 3 * _Q
_NQ = _PAD // _Q


def _body(t0, t1, t2, ids_hbm, split_hbm, out_hbm,
          idx_v, spl_v, sidx_v, spos_v, rows_v, gsem, ssem):
    wid = lax.axis_index("s") * _NC + lax.axis_index("c")
    base = wid * _CHUNK
    pltpu.sync_copy(ids_hbm.at[pl.ds(base, _CHUNK)], idx_v)
    pltpu.sync_copy(split_hbm.at[pl.ds(base, _CHUNK)], spl_v)

    iota = lax.iota(jnp.int32, _L)

    def count(i, c):
        s = spl_v[pl.ds(i * _L, _L)]
        a0, a1 = c
        return (a0 + (s == 0).astype(jnp.int32), a1 + (s == 1).astype(jnp.int32))

    zz = jnp.zeros((_L,), jnp.int32)
    a0, a1 = lax.fori_loop(0, _NV, count, (zz, zz))
    c0 = jnp.sum(a0)
    c1 = jnp.sum(a1)
    c2 = _CHUNK - c0 - c1
    b1 = (c0 + _Q - 1) & ~(_Q - 1)
    b2 = b1 + ((c1 + _Q - 1) & ~(_Q - 1))
    b3 = b2 + ((c2 + _Q - 1) & ~(_Q - 1))

    def place(i, o):
        o0, o1, o2 = o
        s = spl_v[pl.ds(i * _L, _L)]
        ids = idx_v[pl.ds(i * _L, _L)]
        pos = base + i * _L + iota
        m0 = (s == 0).astype(jnp.int32)
        m1 = (s == 1).astype(jnp.int32)
        m2 = (s == 2).astype(jnp.int32)
        r0 = plsc.cumsum(m0)
        r1 = plsc.cumsum(m1)
        r2 = plsc.cumsum(m2)
        slot = (m0 * (o0 + r0 - 1) + m1 * (o1 + r1 - 1) + m2 * (o2 + r2 - 1))
        plsc.store_scatter(sidx_v, [slot], ids)
        plsc.store_scatter(spos_v, [slot >> _QSH, slot & (_Q - 1)], pos)
        return (o0 + r0[_L - 1], o1 + r1[_L - 1], o2 + r2[_L - 1])

    o0, o1, o2 = lax.fori_loop(0, _NV, place, (jnp.int32(0), b1, b2))

    for ot, bt, bn in ((o0, jnp.int32(0), b1), (o1, b1, b2), (o2, b2, b3)):
        first_idx = sidx_v[pl.ds(bt, _L)][0]
        first_pos = spos_v[bt >> _QSH, pl.ds(0, _L)][0]
        fi = jnp.full((_L,), first_idx, jnp.int32)
        fp = jnp.full((_L,), first_pos, jnp.int32)
        for j in range(_Q // _L):
            slots = ot + j * _L + iota
            mask = slots < bn
            plsc.store_scatter(sidx_v, [slots], fi, mask=mask)
            plsc.store_scatter(spos_v, [slots >> _QSH, slots & (_Q - 1)],
                               fp, mask=mask)

    for tab, qs, qe in ((t0, jnp.int32(0), b1 // _Q), (t1, b1 // _Q, b2 // _Q),
                        (t2, b2 // _Q, b3 // _Q)):
        def fire(q, _, tab=tab):
            pltpu.async_copy(tab.at[sidx_v.at[pl.ds(q * _Q, _Q)]],
                             rows_v.at[pl.ds(q * _Q, _Q)], gsem)
            return _
        lax.fori_loop(qs, qe, fire, 0)

    def drain_g(q, _):
        pltpu.make_async_copy(t0.at[sidx_v.at[pl.ds(0, _Q)]],
                              rows_v.at[pl.ds(0, _Q)], gsem).wait()
        return _

    def fire_s(q, _):
        pltpu.async_copy(rows_v.at[pl.ds(q * _Q, _Q)],
                         out_hbm.at[spos_v.at[q]], ssem)
        return _

    lax.fori_loop(0, b2 // _Q, drain_g, 0)
    lax.fori_loop(0, b2 // _Q, fire_s, 0)
    lax.fori_loop(b2 // _Q, b3 // _Q, drain_g, 0)
    lax.fori_loop(b2 // _Q, b3 // _Q, fire_s, 0)

    def drain_s(q, _):
        pltpu.make_async_copy(rows_v.at[pl.ds(0, _Q)],
                              out_hbm.at[spos_v.at[0]], ssem).wait()
        return _
    lax.fori_loop(0, b3 // _Q, drain_s, 0)


@jax.jit
def _run(t0, t1, t2, ids, split):
    mesh = plsc.VectorSubcoreMesh(core_axis_name="c", subcore_axis_name="s")
    return pl.kernel(
        _body,
        mesh=mesh,
        compiler_params=pltpu.CompilerParams(
            needs_layout_passes=False,
            disable_bounds_checks=True,
            disable_semaphore_checks=True,
        ),
        out_type=jax.ShapeDtypeStruct((BATCH, DIM), jnp.float32),
        scratch_types=[
            pltpu.VMEM((_CHUNK,), jnp.int32),
            pltpu.VMEM((_CHUNK,), jnp.int32),
            pltpu.VMEM((_PAD,), jnp.int32),
            pltpu.VMEM((_NQ, _Q), jnp.int32),
            pltpu.VMEM((_PAD, DIM), jnp.float32),
            pltpu.SemaphoreType.DMA,
            pltpu.SemaphoreType.DMA,
        ],
    )(t0, t1, t2, ids, split)


def kernel(query_enc_train, query_enc_dev, query_enc_test, ex_ids, split):
    return _run(query_enc_train, query_enc_dev, query_enc_test,
                ex_ids.astype(jnp.int32), split.astype(jnp.int32))


# speedup vs baseline: 1.1640x; 1.0286x over previous
import jax
import jax.numpy as jnp
from jax import lax
from jax.experimental import pallas as pl
from jax.experimental.pallas import tpu as pltpu
from jax.experimental.pallas import tpu_sc as plsc

VOCAB = 100000
DIM = 128
BATCH = 16384

_info = plsc.get_sparse_core_info()
_NC, _NS, _L = _info.num_cores, _info.num_subcores, _info.num_lanes
_NW = _NC * _NS
_CHUNK = BATCH // _NW
_NV = _CHUNK // _L
_Q = 16
_QSH = _Q.bit_length() - 1
_PAD = _CHUNK + 3 * _Q
_NQ = _PAD // _Q


def _body(t0, t1, t2, ids_hbm, split_hbm, out_hbm,
          idx_v, spl_v, sidx_v, spos_v, rows_v, gsem, ssem):
    wid = lax.axis_index("s") * _NC + lax.axis_index("c")
    base = wid * _CHUNK
    pltpu.sync_copy(ids_hbm.at[pl.ds(base, _CHUNK)], idx_v)
    pltpu.sync_copy(split_hbm.at[pl.ds(base, _CHUNK)], spl_v)

    iota = lax.iota(jnp.int32, _L)

    def count(i, c):
        s = spl_v[pl.ds(i * _L, _L)]
        a0, a1 = c
        return (a0 + (s == 0).astype(jnp.int32), a1 + (s == 1).astype(jnp.int32))

    zz = jnp.zeros((_L,), jnp.int32)
    a0, a1 = lax.fori_loop(0, _NV, count, (zz, zz))
    c0 = jnp.sum(a0)
    c1 = jnp.sum(a1)
    c2 = _CHUNK - c0 - c1
    b1 = (c0 + _Q - 1) & ~(_Q - 1)
    b2 = b1 + ((c1 + _Q - 1) & ~(_Q - 1))
    b3 = b2 + ((c2 + _Q - 1) & ~(_Q - 1))

    def place(i, o):
        o0, o1, o2 = o
        s = spl_v[pl.ds(i * _L, _L)]
        ids = idx_v[pl.ds(i * _L, _L)]
        pos = base + i * _L + iota
        m0 = (s == 0).astype(jnp.int32)
        m1 = (s == 1).astype(jnp.int32)
        m2 = (s == 2).astype(jnp.int32)
        r0 = plsc.cumsum(m0)
        r1 = plsc.cumsum(m1)
        r2 = plsc.cumsum(m2)
        slot = (m0 * (o0 + r0 - 1) + m1 * (o1 + r1 - 1) + m2 * (o2 + r2 - 1))
        plsc.store_scatter(sidx_v, [slot], ids)
        plsc.store_scatter(spos_v, [slot >> _QSH, slot & (_Q - 1)], pos)
        return (o0 + r0[_L - 1], o1 + r1[_L - 1], o2 + r2[_L - 1])

    o0, o1, o2 = lax.fori_loop(0, _NV, place, (jnp.int32(0), b1, b2))

    for ot, bt, bn in ((o0, jnp.int32(0), b1), (o1, b1, b2), (o2, b2, b3)):
        first_idx = sidx_v[pl.ds(bt, _L)][0]
        first_pos = spos_v[bt >> _QSH, pl.ds(0, _L)][0]
        fi = jnp.full((_L,), first_idx, jnp.int32)
        fp = jnp.full((_L,), first_pos, jnp.int32)
        for j in range(_Q // _L):
            slots = ot + j * _L + iota
            mask = slots < bn
            plsc.store_scatter(sidx_v, [slots], fi, mask=mask)
            plsc.store_scatter(spos_v, [slots >> _QSH, slots & (_Q - 1)],
                               fp, mask=mask)

    for tab, qs, qe in ((t0, jnp.int32(0), b1 // _Q), (t1, b1 // _Q, b2 // _Q),
                        (t2, b2 // _Q, b3 // _Q)):
        def fire(q, _, tab=tab):
            pltpu.async_copy(tab.at[sidx_v.at[pl.ds(q * _Q, _Q)]],
                             rows_v.at[pl.ds(q * _Q, _Q)], gsem)
            return _
        lax.fori_loop(qs, qe, fire, 0)

    def drain_g(q, _):
        pltpu.make_async_copy(t0.at[sidx_v.at[pl.ds(0, _Q)]],
                              rows_v.at[pl.ds(0, _Q)], gsem).wait()
        return _
    lax.fori_loop(0, b3 // _Q, drain_g, 0)

    def fire_s(q, _):
        pltpu.async_copy(rows_v.at[pl.ds(q * _Q, _Q)],
                         out_hbm.at[spos_v.at[q]], ssem)
        return _
    lax.fori_loop(0, b3 // _Q, fire_s, 0)

    def drain_s(q, _):
        pltpu.make_async_copy(rows_v.at[pl.ds(0, _Q)],
                              out_hbm.at[spos_v.at[0]], ssem).wait()
        return _
    lax.fori_loop(0, b3 // _Q, drain_s, 0)


@jax.jit
def _run(t0, t1, t2, ids, split):
    mesh = plsc.VectorSubcoreMesh(core_axis_name="c", subcore_axis_name="s")
    return pl.kernel(
        _body,
        mesh=mesh,
        compiler_params=pltpu.CompilerParams(
            needs_layout_passes=False,
            disable_bounds_checks=True,
            disable_semaphore_checks=True,
        ),
        out_type=jax.ShapeDtypeStruct((BATCH, DIM), jnp.float32),
        scratch_types=[
            pltpu.VMEM((_CHUNK,), jnp.int32),
            pltpu.VMEM((_CHUNK,), jnp.int32),
            pltpu.VMEM((_PAD,), jnp.int32),
            pltpu.VMEM((_NQ, _Q), jnp.int32),
            pltpu.VMEM((_PAD, DIM), jnp.float32),
            pltpu.SemaphoreType.DMA,
            pltpu.SemaphoreType.DMA,
        ],
    )(t0, t1, t2, ids, split)


def kernel(query_enc_train, query_enc_dev, query_enc_test, ex_ids, split):
    return _run(query_enc_train, query_enc_dev, query_enc_test,
                ex_ids.astype(jnp.int32), split.astype(jnp.int32))
